# Initial kernel scaffold; baseline (speedup 1.0000x reference)
import jax
import jax.numpy as jnp
from jax.experimental import pallas as pl


def kernel(x, edge_index, W, b, gamma, beta):
    raise NotImplementedError("write your pallas kernel here")


# speedup vs baseline: 3.7762x; 3.7762x over previous
import functools

import jax
import jax.numpy as jnp
from jax import lax
from jax.experimental import pallas as pl
from jax.experimental.pallas import tpu as pltpu
from jax.experimental.pallas import tpu_sc as plsc

N = 10000
E = 320000
D = 128
H = 128

NC = 2
NS = 16
NW = NC * NS
L = 16

NP = 10240
RPT = NP // NS
EPW = E // NW
C = 80
NCHUNK = EPW // C
HL = H // L

_mesh = plsc.VectorSubcoreMesh(
    core_axis_name="c", subcore_axis_name="s", num_cores=NC, num_subcores=NS)


@functools.partial(
    pl.kernel,
    out_type=[
        jax.ShapeDtypeStruct((E, H), jnp.float32),
        jax.ShapeDtypeStruct((NW, H), jnp.float32),
        jax.ShapeDtypeStruct((NW, H), jnp.float32),
        jax.ShapeDtypeStruct((NC, NP), jnp.float32),
    ],
    mesh=_mesh,
    scratch_types=[
        pltpu.VMEM((C,), jnp.int32),
        pltpu.VMEM((C,), jnp.int32),
        pltpu.VMEM((C, H), jnp.float32),
        pltpu.VMEM((C, H), jnp.float32),
        pltpu.VMEM((C, H), jnp.float32),
        pltpu.VMEM((C,), jnp.float32),
        pltpu.VMEM((H,), jnp.float32),
        pltpu.VMEM((RPT,), jnp.float32),
        pltpu.VMEM_SHARED((NP,), jnp.float32),
        pltpu.SemaphoreType.DMA,
        pltpu.SemaphoreType.DMA,
    ],
)
def _sc_pass1(dst_hbm, src_hbm, p1_hbm, p2_hbm,
              mraw_hbm, s1_hbm, s2_hbm, cnt_hbm,
              didx, sidx, r1, r2, mbuf, ones_v, stat_v, zbuf, cnt_sp,
              sem1, sem2):
    cid = lax.axis_index("c")
    sid = lax.axis_index("s")
    wid = sid * NC + cid

    zero16 = jnp.zeros((L,), jnp.float32)
    for t in range(RPT // L):
        zbuf[pl.ds(t * L, L)] = zero16
    for t in range(C // L):
        ones_v[pl.ds(t * L, L)] = jnp.full((L,), 1.0, jnp.float32)
    pltpu.sync_copy(zbuf, cnt_sp.at[pl.ds(sid * RPT, RPT)])
    plsc.subcore_barrier()

    def chunk_body(k, accs):
        base = wid * EPW + k * C
        pltpu.sync_copy(dst_hbm.at[pl.ds(base, C)], didx)
        pltpu.sync_copy(src_hbm.at[pl.ds(base, C)], sidx)
        g1 = pltpu.async_copy(p1_hbm.at[didx], r1, sem1)
        g2 = pltpu.async_copy(p2_hbm.at[sidx], r2, sem2)
        g1.wait()
        g2.wait()

        def row_body(i, a):
            a = list(a)
            for j in range(HL):
                sl = pl.ds(j * L, L)
                s = r1[i, sl] + r2[i, sl]
                mbuf[i, sl] = s
                a[2 * j] = a[2 * j] + s
                a[2 * j + 1] = a[2 * j + 1] + s * s
            return tuple(a)

        accs = lax.fori_loop(0, C, row_body, accs)
        pltpu.sync_copy(mbuf, mraw_hbm.at[pl.ds(base, C)])
        pltpu.sync_copy(ones_v, cnt_sp.at[didx], add=True)
        return accs

    accs = lax.fori_loop(
        0, NCHUNK, chunk_body,
        tuple(jnp.zeros((L,), jnp.float32) for _ in range(2 * HL)))

    for j in range(HL):
        stat_v[pl.ds(j * L, L)] = accs[2 * j]
    pltpu.sync_copy(stat_v, s1_hbm.at[wid])
    for j in range(HL):
        stat_v[pl.ds(j * L, L)] = accs[2 * j + 1]
    pltpu.sync_copy(stat_v, s2_hbm.at[wid])

    plsc.subcore_barrier()
    pltpu.sync_copy(cnt_sp.at[pl.ds(sid * RPT, RPT)],
                    cnt_hbm.at[cid, pl.ds(sid * RPT, RPT)])


@functools.partial(
    pl.kernel,
    out_type=[
        jax.ShapeDtypeStruct((E, H), jnp.float32),
        jax.ShapeDtypeStruct((NC, NP, H), jnp.float32),
    ],
    mesh=_mesh,
    scratch_types=[
        pltpu.VMEM((C,), jnp.int32),
        pltpu.VMEM((C, H), jnp.float32),
        pltpu.VMEM((C, H), jnp.float32),
        pltpu.VMEM((H,), jnp.float32),
        pltpu.VMEM((H,), jnp.float32),
        pltpu.VMEM_SHARED((NP, H), jnp.float32),
        pltpu.SemaphoreType.DMA,
    ],
)
def _sc_pass2(dst_hbm, mraw_hbm, scale_hbm, shift_hbm,
              msg_hbm, agg_hbm,
              didx, min_v, mout_v, sc_v, sh_v, acc_sp, sem1):
    cid = lax.axis_index("c")
    sid = lax.axis_index("s")
    wid = sid * NC + cid

    pltpu.sync_copy(scale_hbm, sc_v)
    pltpu.sync_copy(shift_hbm, sh_v)
    scs = [sc_v[pl.ds(j * L, L)] for j in range(HL)]
    shs = [sh_v[pl.ds(j * L, L)] for j in range(HL)]

    zero16 = jnp.zeros((L,), jnp.float32)

    def zrow(i, carry):
        for j in range(HL):
            min_v[i, pl.ds(j * L, L)] = zero16
        return carry

    lax.fori_loop(0, C, zrow, 0)
    for t in range(RPT // C):
        pltpu.sync_copy(min_v, acc_sp.at[pl.ds(sid * RPT + t * C, C)])
    plsc.subcore_barrier()

    def chunk_body(k, carry):
        base = wid * EPW + k * C
        pltpu.sync_copy(dst_hbm.at[pl.ds(base, C)], didx)
        pltpu.sync_copy(mraw_hbm.at[pl.ds(base, C)], min_v)

        def row_body(i, c2):
            for j in range(HL):
                sl = pl.ds(j * L, L)
                mout_v[i, sl] = jnp.maximum(min_v[i, sl] * scs[j] + shs[j], 0.0)
            return c2

        lax.fori_loop(0, C, row_body, 0)
        pltpu.sync_copy(mout_v, msg_hbm.at[pl.ds(base, C)])
        pltpu.sync_copy(mout_v, acc_sp.at[didx], add=True)
        return carry

    lax.fori_loop(0, NCHUNK, chunk_body, 0)

    plsc.subcore_barrier()
    pltpu.sync_copy(acc_sp.at[pl.ds(sid * RPT, RPT)],
                    agg_hbm.at[cid, pl.ds(sid * RPT, RPT)])


def _proj_body(x_ref, w1_ref, w2_ref, o1_ref, o2_ref):
    xb = x_ref[...]
    o1_ref[...] = jnp.dot(xb, w1_ref[...], preferred_element_type=jnp.float32)
    o2_ref[...] = jnp.dot(xb, w2_ref[...], preferred_element_type=jnp.float32)


def _project(x, w1t, w2t):
    nb = 10
    bs = N // nb
    return pl.pallas_call(
        _proj_body,
        grid=(nb,),
        in_specs=[
            pl.BlockSpec((bs, D), lambda i: (i, 0)),
            pl.BlockSpec((D, H), lambda i: (0, 0)),
            pl.BlockSpec((D, H), lambda i: (0, 0)),
        ],
        out_specs=[
            pl.BlockSpec((bs, H), lambda i: (i, 0)),
            pl.BlockSpec((bs, H), lambda i: (i, 0)),
        ],
        out_shape=[
            jax.ShapeDtypeStruct((N, H), jnp.float32),
            jax.ShapeDtypeStruct((N, H), jnp.float32),
        ],
    )(x, w1t, w2t)


def _mean_body(a0_ref, a1_ref, c0_ref, c1_ref, o_ref):
    cnt = jnp.maximum(c0_ref[...] + c1_ref[...], 1.0)
    o_ref[...] = (a0_ref[...] + a1_ref[...]) / cnt


def _finalize(a0, a1, c0, c1):
    nb = 10
    bs = N // nb
    return pl.pallas_call(
        _mean_body,
        grid=(nb,),
        in_specs=[
            pl.BlockSpec((bs, H), lambda i: (i, 0)),
            pl.BlockSpec((bs, H), lambda i: (i, 0)),
            pl.BlockSpec((bs, 1), lambda i: (i, 0)),
            pl.BlockSpec((bs, 1), lambda i: (i, 0)),
        ],
        out_specs=pl.BlockSpec((bs, H), lambda i: (i, 0)),
        out_shape=jax.ShapeDtypeStruct((N, H), jnp.float32),
    )(a0, a1, c0, c1)


@jax.jit
def kernel(x, edge_index, W, b, gamma, beta):
    src = edge_index[0].astype(jnp.int32)
    dst = edge_index[1].astype(jnp.int32)
    w1t = W[:, :D].T
    w2t = W[:, D:].T

    p1, p2 = _project(x, w1t, w2t)

    mraw, s1p, s2p, cntp = _sc_pass1(dst, src, p1, p2)

    s1 = jnp.sum(s1p, axis=0)
    s2 = jnp.sum(s2p, axis=0)
    mean_s = s1 / E
    var = s2 / E - mean_s * mean_s
    scale = gamma / jnp.sqrt(var + 1e-5)
    shift = beta - mean_s * scale

    msg, aggp = _sc_pass2(dst, mraw, scale, shift)

    out = _finalize(aggp[0, :N], aggp[1, :N],
                    cntp[0, :N, None], cntp[1, :N, None])
    return out, msg


# speedup vs baseline: 6.5132x; 1.7248x over previous
import functools

import jax
import jax.numpy as jnp
from jax import lax
from jax.experimental import pallas as pl
from jax.experimental.pallas import tpu as pltpu
from jax.experimental.pallas import tpu_sc as plsc

N = 10000
E = 320000
D = 128
H = 128

NC = 2
NS = 16
NW = NC * NS
L = 16

NP = 10240
RPT = NP // NS
EPW = E // NW
C = 80
NCHUNK = EPW // C
HL = H // L

_mesh = plsc.VectorSubcoreMesh(
    core_axis_name="c", subcore_axis_name="s", num_cores=NC, num_subcores=NS)


@functools.partial(
    pl.kernel,
    out_type=[
        jax.ShapeDtypeStruct((E, H), jnp.float32),
        jax.ShapeDtypeStruct((NW, H), jnp.float32),
        jax.ShapeDtypeStruct((NW, H), jnp.float32),
        jax.ShapeDtypeStruct((NC, NP), jnp.float32),
    ],
    mesh=_mesh,
    scratch_types=[
        [pltpu.VMEM((C,), jnp.int32)] * 2,
        [pltpu.VMEM((C,), jnp.int32)] * 2,
        [pltpu.VMEM((C, H), jnp.float32)] * 2,
        [pltpu.VMEM((C, H), jnp.float32)] * 2,
        [pltpu.VMEM((C, H), jnp.float32)] * 2,
        pltpu.VMEM((C,), jnp.float32),
        pltpu.VMEM((H,), jnp.float32),
        pltpu.VMEM((RPT,), jnp.float32),
        pltpu.VMEM_SHARED((NP,), jnp.float32),
        [pltpu.SemaphoreType.DMA] * 2,
        [pltpu.SemaphoreType.DMA] * 2,
        [pltpu.SemaphoreType.DMA] * 2,
    ],
)
def _sc_pass1(dst_hbm, src_hbm, p1_hbm, p2_hbm,
              mraw_hbm, s1_hbm, s2_hbm, cnt_hbm,
              didx, sidx, r1, r2, mbuf, ones_v, stat_v, zbuf, cnt_sp,
              g1sem, g2sem, wsem):
    cid = lax.axis_index("c")
    sid = lax.axis_index("s")
    wid = sid * NC + cid
    ebase = wid * EPW

    zero16 = jnp.zeros((L,), jnp.float32)
    for t in range(RPT // L):
        zbuf[pl.ds(t * L, L)] = zero16
    for t in range(C // L):
        ones_v[pl.ds(t * L, L)] = jnp.full((L,), 1.0, jnp.float32)
    pltpu.sync_copy(zbuf, cnt_sp.at[pl.ds(sid * RPT, RPT)])
    plsc.subcore_barrier()

    def fetch(k, b):
        pltpu.sync_copy(dst_hbm.at[pl.ds(ebase + k * C, C)], didx[b])
        pltpu.sync_copy(src_hbm.at[pl.ds(ebase + k * C, C)], sidx[b])
        pltpu.async_copy(p1_hbm.at[didx[b]], r1[b], g1sem[b])
        pltpu.async_copy(p2_hbm.at[sidx[b]], r2[b], g2sem[b])

    def step(k, b, accs, wait_write, prefetch):
        pltpu.make_async_copy(p1_hbm.at[didx[b]], r1[b], g1sem[b]).wait()
        pltpu.make_async_copy(p2_hbm.at[sidx[b]], r2[b], g2sem[b]).wait()
        if wait_write is True:
            pltpu.make_async_copy(
                mbuf[b], mraw_hbm.at[pl.ds(0, C)], wsem[b]).wait()
        elif wait_write is not None:
            @pl.when(wait_write)
            def _():
                pltpu.make_async_copy(
                    mbuf[b], mraw_hbm.at[pl.ds(0, C)], wsem[b]).wait()

        def row_body(i, a):
            a = list(a)
            for j in range(HL):
                sl = pl.ds(j * L, L)
                s = r1[b][i, sl] + r2[b][i, sl]
                mbuf[b][i, sl] = s
                a[2 * j] = a[2 * j] + s
                a[2 * j + 1] = a[2 * j + 1] + s * s
            return tuple(a)

        accs = lax.fori_loop(0, C, row_body, accs)
        pltpu.async_copy(mbuf[b], mraw_hbm.at[pl.ds(ebase + k * C, C)],
                         wsem[b])
        pltpu.sync_copy(ones_v, cnt_sp.at[didx[b]], add=True)
        if prefetch is not None:
            @pl.when(prefetch)
            def _():
                fetch(k + 2, b)
        return accs

    fetch(0, 0)
    fetch(1, 1)

    def gbody(g, accs):
        for b in range(2):
            k = 2 * g + b
            accs = step(k, b, accs, wait_write=(g >= 1),
                        prefetch=(k + 2 < NCHUNK))
        return accs

    accs = lax.fori_loop(
        0, NCHUNK // 2, gbody,
        tuple(jnp.zeros((L,), jnp.float32) for _ in range(2 * HL)))
    accs = step(NCHUNK - 1, 0, accs, wait_write=True, prefetch=None)
    pltpu.make_async_copy(mbuf[0], mraw_hbm.at[pl.ds(0, C)], wsem[0]).wait()
    pltpu.make_async_copy(mbuf[1], mraw_hbm.at[pl.ds(0, C)], wsem[1]).wait()

    for j in range(HL):
        stat_v[pl.ds(j * L, L)] = accs[2 * j]
    pltpu.sync_copy(stat_v, s1_hbm.at[wid])
    for j in range(HL):
        stat_v[pl.ds(j * L, L)] = accs[2 * j + 1]
    pltpu.sync_copy(stat_v, s2_hbm.at[wid])

    plsc.subcore_barrier()
    pltpu.sync_copy(cnt_sp.at[pl.ds(sid * RPT, RPT)],
                    cnt_hbm.at[cid, pl.ds(sid * RPT, RPT)])


@functools.partial(
    pl.kernel,
    out_type=[
        jax.ShapeDtypeStruct((E, H), jnp.float32),
        jax.ShapeDtypeStruct((NC, NP, H), jnp.float32),
    ],
    mesh=_mesh,
    scratch_types=[
        [pltpu.VMEM((C,), jnp.int32)] * 2,
        [pltpu.VMEM((C, H), jnp.float32)] * 2,
        [pltpu.VMEM((C, H), jnp.float32)] * 2,
        pltpu.VMEM((H,), jnp.float32),
        pltpu.VMEM((H,), jnp.float32),
        pltpu.VMEM_SHARED((NP, H), jnp.float32),
        [pltpu.SemaphoreType.DMA] * 2,
        [pltpu.SemaphoreType.DMA] * 2,
    ],
)
def _sc_pass2(dst_hbm, mraw_hbm, scale_hbm, shift_hbm,
              msg_hbm, agg_hbm,
              didx, min_v, mout_v, sc_v, sh_v, acc_sp, rsem, wsem):
    cid = lax.axis_index("c")
    sid = lax.axis_index("s")
    wid = sid * NC + cid
    ebase = wid * EPW

    pltpu.sync_copy(scale_hbm, sc_v)
    pltpu.sync_copy(shift_hbm, sh_v)
    scs = [sc_v[pl.ds(j * L, L)] for j in range(HL)]
    shs = [sh_v[pl.ds(j * L, L)] for j in range(HL)]

    zero16 = jnp.zeros((L,), jnp.float32)

    def zrow(i, carry):
        for j in range(HL):
            min_v[0][i, pl.ds(j * L, L)] = zero16
        return carry

    lax.fori_loop(0, C, zrow, 0)
    for t in range(RPT // C):
        pltpu.sync_copy(min_v[0], acc_sp.at[pl.ds(sid * RPT + t * C, C)])
    plsc.subcore_barrier()

    def fetch(k, b):
        pltpu.sync_copy(dst_hbm.at[pl.ds(ebase + k * C, C)], didx[b])
        pltpu.async_copy(mraw_hbm.at[pl.ds(ebase + k * C, C)], min_v[b],
                         rsem[b])

    def step(k, b, wait_write, prefetch):
        pltpu.make_async_copy(
            mraw_hbm.at[pl.ds(0, C)], min_v[b], rsem[b]).wait()
        if wait_write is True:
            pltpu.make_async_copy(
                mout_v[b], msg_hbm.at[pl.ds(0, C)], wsem[b]).wait()
        elif wait_write is not None:
            @pl.when(wait_write)
            def _():
                pltpu.make_async_copy(
                    mout_v[b], msg_hbm.at[pl.ds(0, C)], wsem[b]).wait()

        def row_body(i, c2):
            for j in range(HL):
                sl = pl.ds(j * L, L)
                mout_v[b][i, sl] = jnp.maximum(
                    min_v[b][i, sl] * scs[j] + shs[j], 0.0)
            return c2

        lax.fori_loop(0, C, row_body, 0)
        pltpu.async_copy(mout_v[b], msg_hbm.at[pl.ds(ebase + k * C, C)],
                         wsem[b])
        pltpu.sync_copy(mout_v[b], acc_sp.at[didx[b]], add=True)
        if prefetch is not None:
            @pl.when(prefetch)
            def _():
                fetch(k + 2, b)

    fetch(0, 0)
    fetch(1, 1)

    def gbody(g, carry):
        for b in range(2):
            k = 2 * g + b
            step(k, b, wait_write=(g >= 1), prefetch=(k + 2 < NCHUNK))
        return carry

    lax.fori_loop(0, NCHUNK // 2, gbody, 0)
    step(NCHUNK - 1, 0, wait_write=True, prefetch=None)
    pltpu.make_async_copy(mout_v[0], msg_hbm.at[pl.ds(0, C)], wsem[0]).wait()
    pltpu.make_async_copy(mout_v[1], msg_hbm.at[pl.ds(0, C)], wsem[1]).wait()

    plsc.subcore_barrier()
    pltpu.sync_copy(acc_sp.at[pl.ds(sid * RPT, RPT)],
                    agg_hbm.at[cid, pl.ds(sid * RPT, RPT)])


def _proj_body(x_ref, w1_ref, w2_ref, o1_ref, o2_ref):
    xb = x_ref[...]
    o1_ref[...] = jnp.dot(xb, w1_ref[...], preferred_element_type=jnp.float32)
    o2_ref[...] = jnp.dot(xb, w2_ref[...], preferred_element_type=jnp.float32)


def _project(x, w1t, w2t):
    nb = 10
    bs = N // nb
    return pl.pallas_call(
        _proj_body,
        grid=(nb,),
        in_specs=[
            pl.BlockSpec((bs, D), lambda i: (i, 0)),
            pl.BlockSpec((D, H), lambda i: (0, 0)),
            pl.BlockSpec((D, H), lambda i: (0, 0)),
        ],
        out_specs=[
            pl.BlockSpec((bs, H), lambda i: (i, 0)),
            pl.BlockSpec((bs, H), lambda i: (i, 0)),
        ],
        out_shape=[
            jax.ShapeDtypeStruct((N, H), jnp.float32),
            jax.ShapeDtypeStruct((N, H), jnp.float32),
        ],
    )(x, w1t, w2t)


def _mean_body(a0_ref, a1_ref, c0_ref, c1_ref, o_ref):
    cnt = jnp.maximum(c0_ref[...] + c1_ref[...], 1.0)
    o_ref[...] = (a0_ref[...] + a1_ref[...]) / cnt


def _finalize(a0, a1, c0, c1):
    nb = 10
    bs = N // nb
    return pl.pallas_call(
        _mean_body,
        grid=(nb,),
        in_specs=[
            pl.BlockSpec((bs, H), lambda i: (i, 0)),
            pl.BlockSpec((bs, H), lambda i: (i, 0)),
            pl.BlockSpec((bs, 1), lambda i: (i, 0)),
            pl.BlockSpec((bs, 1), lambda i: (i, 0)),
        ],
        out_specs=pl.BlockSpec((bs, H), lambda i: (i, 0)),
        out_shape=jax.ShapeDtypeStruct((N, H), jnp.float32),
    )(a0, a1, c0, c1)


@jax.jit
def kernel(x, edge_index, W, b, gamma, beta):
    src = edge_index[0].astype(jnp.int32)
    dst = edge_index[1].astype(jnp.int32)
    w1t = W[:, :D].T
    w2t = W[:, D:].T

    p1, p2 = _project(x, w1t, w2t)

    mraw, s1p, s2p, cntp = _sc_pass1(dst, src, p1, p2)

    s1 = jnp.sum(s1p, axis=0)
    s2 = jnp.sum(s2p, axis=0)
    mean_s = s1 / E
    var = s2 / E - mean_s * mean_s
    scale = gamma / jnp.sqrt(var + 1e-5)
    shift = beta - mean_s * scale

    msg, aggp = _sc_pass2(dst, mraw, scale, shift)

    out = _finalize(aggp[0, :N], aggp[1, :N],
                    cntp[0, :N, None], cntp[1, :N, None])
    return out, msg


# speedup vs baseline: 8.2300x; 1.2636x over previous
import functools

import jax
import jax.numpy as jnp
from jax import lax
from jax.experimental import pallas as pl
from jax.experimental.pallas import tpu as pltpu
from jax.experimental.pallas import tpu_sc as plsc

N = 10000
E = 320000
D = 128
H = 128

NC = 2
NS = 16
NW = NC * NS
L = 16

NP = 10240
RPT = NP // NS
EPW = E // NW
C = 80
NCHUNK = EPW // C
G = NCHUNK // 4
HL = H // L

_mesh = plsc.VectorSubcoreMesh(
    core_axis_name="c", subcore_axis_name="s", num_cores=NC, num_subcores=NS)


@functools.partial(
    pl.kernel,
    out_type=[
        jax.ShapeDtypeStruct((E, H), jnp.float32),
        jax.ShapeDtypeStruct((NW, H), jnp.float32),
        jax.ShapeDtypeStruct((NW, H), jnp.float32),
        jax.ShapeDtypeStruct((NC, NP), jnp.float32),
    ],
    mesh=_mesh,
    scratch_types=[
        [pltpu.VMEM((C,), jnp.int32)] * 4,
        [pltpu.VMEM((C,), jnp.int32)] * 4,
        [pltpu.VMEM((C, H), jnp.float32)] * 4,
        [pltpu.VMEM((C, H), jnp.float32)] * 4,
        [pltpu.VMEM((C, H), jnp.float32)] * 2,
        pltpu.VMEM((C,), jnp.float32),
        pltpu.VMEM((H,), jnp.float32),
        pltpu.VMEM((RPT,), jnp.float32),
        pltpu.VMEM_SHARED((NP,), jnp.float32),
        [pltpu.SemaphoreType.DMA] * 4,
        [pltpu.SemaphoreType.DMA] * 4,
        [pltpu.SemaphoreType.DMA] * 2,
    ],
)
def _sc_pass1(dst_hbm, src_hbm, p1_hbm, p2_hbm,
              mraw_hbm, s1_hbm, s2_hbm, cnt_hbm,
              didx, sidx, r1, r2, mbuf, ones_v, stat_v, zbuf, cnt_sp,
              isem, gsem, wsem):
    cid = lax.axis_index("c")
    sid = lax.axis_index("s")
    wid = sid * NC + cid
    ebase = wid * EPW

    zero16 = jnp.zeros((L,), jnp.float32)
    for t in range(RPT // L):
        zbuf[pl.ds(t * L, L)] = zero16
    for t in range(C // L):
        ones_v[pl.ds(t * L, L)] = jnp.full((L,), 1.0, jnp.float32)
    pltpu.sync_copy(zbuf, cnt_sp.at[pl.ds(sid * RPT, RPT)])
    plsc.subcore_barrier()

    def idx_load(k, s):
        pltpu.async_copy(dst_hbm.at[pl.ds(ebase + k * C, C)], didx[s],
                         isem[s])
        pltpu.async_copy(src_hbm.at[pl.ds(ebase + k * C, C)], sidx[s],
                         isem[s])

    def idx_wait(s):
        pltpu.make_async_copy(dst_hbm.at[pl.ds(0, C)], didx[s],
                              isem[s]).wait()
        pltpu.make_async_copy(src_hbm.at[pl.ds(0, C)], sidx[s],
                              isem[s]).wait()

    def gather(s):
        pltpu.async_copy(p1_hbm.at[didx[s]], r1[s], gsem[s])
        pltpu.async_copy(p2_hbm.at[sidx[s]], r2[s], gsem[s])

    def gather_wait(s):
        pltpu.make_async_copy(p1_hbm.at[didx[s]], r1[s], gsem[s]).wait()
        pltpu.make_async_copy(p2_hbm.at[sidx[s]], r2[s], gsem[s]).wait()

    def step(k, u4, u2, g, accs, tail=False):
        gather_wait(u4)
        if not tail:
            s2 = (u4 + 2) % 4

            def pg():
                idx_wait(s2)
                gather(s2)

            if u4 == 3:
                pl.when(g < G - 1)(pg)
            else:
                pg()
        def ww():
            pltpu.make_async_copy(
                mbuf[u2], mraw_hbm.at[pl.ds(0, C)], wsem[u2]).wait()

        if tail or u4 >= 2:
            ww()
        else:
            pl.when(g >= 1)(ww)

        def row_body(i, a):
            a = list(a)
            for j in range(HL):
                sl = pl.ds(j * L, L)
                s = r1[u4][i, sl] + r2[u4][i, sl]
                mbuf[u2][i, sl] = s
                a[2 * j] = a[2 * j] + s
                a[2 * j + 1] = a[2 * j + 1] + s * s
            return tuple(a)

        accs = lax.fori_loop(0, C, row_body, accs)
        pltpu.async_copy(mbuf[u2], mraw_hbm.at[pl.ds(ebase + k * C, C)],
                         wsem[u2])
        pltpu.sync_copy(ones_v, cnt_sp.at[didx[u4]], add=True)
        if not tail:
            def pi():
                idx_load(k + 4, u4)

            if u4 == 0:
                pi()
            else:
                pl.when(g < G - 1)(pi)
        return accs

    for s in range(4):
        idx_load(s, s)
    idx_wait(0)
    gather(0)
    idx_wait(1)
    gather(1)

    def gbody(g, accs):
        for u in range(4):
            accs = step(4 * g + u, u, u % 2, g, accs)
        return accs

    accs = lax.fori_loop(
        0, G, gbody,
        tuple(jnp.zeros((L,), jnp.float32) for _ in range(2 * HL)))
    accs = step(NCHUNK - 1, 0, 0, G, accs, tail=True)
    pltpu.make_async_copy(mbuf[1], mraw_hbm.at[pl.ds(0, C)], wsem[1]).wait()
    pltpu.make_async_copy(mbuf[0], mraw_hbm.at[pl.ds(0, C)], wsem[0]).wait()

    for j in range(HL):
        stat_v[pl.ds(j * L, L)] = accs[2 * j]
    pltpu.sync_copy(stat_v, s1_hbm.at[wid])
    for j in range(HL):
        stat_v[pl.ds(j * L, L)] = accs[2 * j + 1]
    pltpu.sync_copy(stat_v, s2_hbm.at[wid])

    plsc.subcore_barrier()
    pltpu.sync_copy(cnt_sp.at[pl.ds(sid * RPT, RPT)],
                    cnt_hbm.at[cid, pl.ds(sid * RPT, RPT)])


@functools.partial(
    pl.kernel,
    out_type=[
        jax.ShapeDtypeStruct((E, H), jnp.float32),
        jax.ShapeDtypeStruct((NC, NP, H), jnp.float32),
    ],
    mesh=_mesh,
    scratch_types=[
        [pltpu.VMEM((C,), jnp.int32)] * 4,
        [pltpu.VMEM((C, H), jnp.float32)] * 2,
        [pltpu.VMEM((C, H), jnp.float32)] * 2,
        pltpu.VMEM((H,), jnp.float32),
        pltpu.VMEM((H,), jnp.float32),
        pltpu.VMEM_SHARED((NP, H), jnp.float32),
        [pltpu.SemaphoreType.DMA] * 4,
        [pltpu.SemaphoreType.DMA] * 2,
        [pltpu.SemaphoreType.DMA] * 2,
    ],
)
def _sc_pass2(dst_hbm, mraw_hbm, scale_hbm, shift_hbm,
              msg_hbm, agg_hbm,
              didx, min_v, mout_v, sc_v, sh_v, acc_sp,
              isem, rsem, wsem):
    cid = lax.axis_index("c")
    sid = lax.axis_index("s")
    wid = sid * NC + cid
    ebase = wid * EPW

    pltpu.sync_copy(scale_hbm, sc_v)
    pltpu.sync_copy(shift_hbm, sh_v)
    scs = [sc_v[pl.ds(j * L, L)] for j in range(HL)]
    shs = [sh_v[pl.ds(j * L, L)] for j in range(HL)]

    zero16 = jnp.zeros((L,), jnp.float32)

    def zrow(i, carry):
        for j in range(HL):
            mout_v[0][i, pl.ds(j * L, L)] = zero16
        return carry

    lax.fori_loop(0, C, zrow, 0)
    for t in range(RPT // C):
        pltpu.sync_copy(mout_v[0], acc_sp.at[pl.ds(sid * RPT + t * C, C)])
    plsc.subcore_barrier()

    def idx_load(k, s):
        pltpu.async_copy(dst_hbm.at[pl.ds(ebase + k * C, C)], didx[s],
                         isem[s])

    def idx_wait(s):
        pltpu.make_async_copy(dst_hbm.at[pl.ds(0, C)], didx[s],
                              isem[s]).wait()

    def mread(k, s):
        pltpu.async_copy(mraw_hbm.at[pl.ds(ebase + k * C, C)], min_v[s],
                         rsem[s])

    def mread_wait(s):
        pltpu.make_async_copy(mraw_hbm.at[pl.ds(0, C)], min_v[s],
                              rsem[s]).wait()

    def step(k, u4, u2, g, tail=False):
        mread_wait(u2)
        def ww():
            pltpu.make_async_copy(
                mout_v[u2], msg_hbm.at[pl.ds(0, C)], wsem[u2]).wait()

        if tail or u4 >= 2:
            ww()
        else:
            pl.when(g >= 1)(ww)

        def row_body(i, c2):
            for j in range(HL):
                sl = pl.ds(j * L, L)
                mout_v[u2][i, sl] = jnp.maximum(
                    min_v[u2][i, sl] * scs[j] + shs[j], 0.0)
            return c2

        lax.fori_loop(0, C, row_body, 0)
        idx_wait(u4)
        pltpu.async_copy(mout_v[u2], msg_hbm.at[pl.ds(ebase + k * C, C)],
                         wsem[u2])
        pltpu.sync_copy(mout_v[u2], acc_sp.at[didx[u4]], add=True)
        if not tail:
            s2 = (u4 + 2) % 4

            def pf():
                mread(k + 2, u2)
                idx_load(k + 2, s2)

            if u4 == 3:
                pl.when(g < G - 1)(pf)
            else:
                pf()

    for s in range(2):
        idx_load(s, s)
        mread(s, s)

    def gbody(g, carry):
        for u in range(4):
            step(4 * g + u, u, u % 2, g)
        return carry

    lax.fori_loop(0, G, gbody, 0)
    step(NCHUNK - 1, 0, 0, G, tail=True)
    pltpu.make_async_copy(mout_v[1], msg_hbm.at[pl.ds(0, C)], wsem[1]).wait()
    pltpu.make_async_copy(mout_v[0], msg_hbm.at[pl.ds(0, C)], wsem[0]).wait()

    plsc.subcore_barrier()
    pltpu.sync_copy(acc_sp.at[pl.ds(sid * RPT, RPT)],
                    agg_hbm.at[cid, pl.ds(sid * RPT, RPT)])


def _proj_body(x_ref, w1_ref, w2_ref, o1_ref, o2_ref):
    xb = x_ref[...]
    o1_ref[...] = jnp.dot(xb, w1_ref[...], preferred_element_type=jnp.float32)
    o2_ref[...] = jnp.dot(xb, w2_ref[...], preferred_element_type=jnp.float32)


def _project(x, w1t, w2t):
    nb = 10
    bs = N // nb
    return pl.pallas_call(
        _proj_body,
        grid=(nb,),
        in_specs=[
            pl.BlockSpec((bs, D), lambda i: (i, 0)),
            pl.BlockSpec((D, H), lambda i: (0, 0)),
            pl.BlockSpec((D, H), lambda i: (0, 0)),
        ],
        out_specs=[
            pl.BlockSpec((bs, H), lambda i: (i, 0)),
            pl.BlockSpec((bs, H), lambda i: (i, 0)),
        ],
        out_shape=[
            jax.ShapeDtypeStruct((N, H), jnp.float32),
            jax.ShapeDtypeStruct((N, H), jnp.float32),
        ],
    )(x, w1t, w2t)


def _mean_body(a0_ref, a1_ref, c0_ref, c1_ref, o_ref):
    cnt = jnp.maximum(c0_ref[...] + c1_ref[...], 1.0)
    o_ref[...] = (a0_ref[...] + a1_ref[...]) / cnt


def _finalize(a0, a1, c0, c1):
    nb = 10
    bs = N // nb
    return pl.pallas_call(
        _mean_body,
        grid=(nb,),
        in_specs=[
            pl.BlockSpec((bs, H), lambda i: (i, 0)),
            pl.BlockSpec((bs, H), lambda i: (i, 0)),
            pl.BlockSpec((bs, 1), lambda i: (i, 0)),
            pl.BlockSpec((bs, 1), lambda i: (i, 0)),
        ],
        out_specs=pl.BlockSpec((bs, H), lambda i: (i, 0)),
        out_shape=jax.ShapeDtypeStruct((N, H), jnp.float32),
    )(a0, a1, c0, c1)


@jax.jit
def kernel(x, edge_index, W, b, gamma, beta):
    src = edge_index[0].astype(jnp.int32)
    dst = edge_index[1].astype(jnp.int32)
    w1t = W[:, :D].T
    w2t = W[:, D:].T

    p1, p2 = _project(x, w1t, w2t)

    mraw, s1p, s2p, cntp = _sc_pass1(dst, src, p1, p2)

    s1 = jnp.sum(s1p, axis=0)
    s2 = jnp.sum(s2p, axis=0)
    mean_s = s1 / E
    var = s2 / E - mean_s * mean_s
    scale = gamma / jnp.sqrt(var + 1e-5)
    shift = beta - mean_s * scale

    msg, aggp = _sc_pass2(dst, mraw, scale, shift)

    out = _finalize(aggp[0, :N], aggp[1, :N],
                    cntp[0, :N, None], cntp[1, :N, None])
    return out, msg


# speedup vs baseline: 8.3338x; 1.0126x over previous
import functools

import jax
import jax.numpy as jnp
from jax import lax
from jax.experimental import pallas as pl
from jax.experimental.pallas import tpu as pltpu
from jax.experimental.pallas import tpu_sc as plsc

N = 10000
E = 320000
D = 128
H = 128

NC = 2
NS = 16
NW = NC * NS
L = 16

NP = 10240
RPT = NP // NS
EPW = E // NW
C = 80
NCHUNK = EPW // C
G = NCHUNK // 4
HL = H // L

_mesh = plsc.VectorSubcoreMesh(
    core_axis_name="c", subcore_axis_name="s", num_cores=NC, num_subcores=NS)


@functools.partial(
    pl.kernel,
    out_type=[
        jax.ShapeDtypeStruct((E, H), jnp.float32),
        jax.ShapeDtypeStruct((NW, H), jnp.float32),
        jax.ShapeDtypeStruct((NW, H), jnp.float32),
        jax.ShapeDtypeStruct((NC, NP), jnp.float32),
    ],
    mesh=_mesh,
    scratch_types=[
        [pltpu.VMEM((C,), jnp.int32)] * 4,
        [pltpu.VMEM((C,), jnp.int32)] * 4,
        [pltpu.VMEM((C, H), jnp.float32)] * 4,
        [pltpu.VMEM((C, H), jnp.float32)] * 4,
        [pltpu.VMEM((C, H), jnp.float32)] * 2,
        pltpu.VMEM((C,), jnp.float32),
        pltpu.VMEM((H,), jnp.float32),
        pltpu.VMEM((RPT,), jnp.float32),
        pltpu.VMEM_SHARED((NP,), jnp.float32),
        [pltpu.SemaphoreType.DMA] * 4,
        [pltpu.SemaphoreType.DMA] * 4,
        [pltpu.SemaphoreType.DMA] * 2,
    ],
)
def _sc_pass1(dst_hbm, src_hbm, p1_hbm, p2_hbm,
              mraw_hbm, s1_hbm, s2_hbm, cnt_hbm,
              didx, sidx, r1, r2, mbuf, ones_v, stat_v, zbuf, cnt_sp,
              isem, gsem, wsem):
    cid = lax.axis_index("c")
    sid = lax.axis_index("s")
    wid = sid * NC + cid
    ebase = wid * EPW

    zero16 = jnp.zeros((L,), jnp.float32)
    for t in range(RPT // L):
        zbuf[pl.ds(t * L, L)] = zero16
    for t in range(C // L):
        ones_v[pl.ds(t * L, L)] = jnp.full((L,), 1.0, jnp.float32)
    pltpu.sync_copy(zbuf, cnt_sp.at[pl.ds(sid * RPT, RPT)])
    plsc.subcore_barrier()

    def idx_load(k, s):
        pltpu.async_copy(dst_hbm.at[pl.ds(ebase + k * C, C)], didx[s],
                         isem[s])
        pltpu.async_copy(src_hbm.at[pl.ds(ebase + k * C, C)], sidx[s],
                         isem[s])

    def idx_wait(s):
        pltpu.make_async_copy(dst_hbm.at[pl.ds(0, C)], didx[s],
                              isem[s]).wait()
        pltpu.make_async_copy(src_hbm.at[pl.ds(0, C)], sidx[s],
                              isem[s]).wait()

    def gather(s):
        pltpu.async_copy(p1_hbm.at[didx[s]], r1[s], gsem[s])
        pltpu.async_copy(p2_hbm.at[sidx[s]], r2[s], gsem[s])

    def gather_wait(s):
        pltpu.make_async_copy(p1_hbm.at[didx[s]], r1[s], gsem[s]).wait()
        pltpu.make_async_copy(p2_hbm.at[sidx[s]], r2[s], gsem[s]).wait()

    def step(k, u4, u2, g, accs, tail=False):
        gather_wait(u4)
        if not tail:
            s2 = (u4 + 2) % 4

            def pg():
                idx_wait(s2)
                gather(s2)

            if u4 == 3:
                pl.when(g < G - 1)(pg)
            else:
                pg()
        def ww():
            pltpu.make_async_copy(
                mbuf[u2], mraw_hbm.at[pl.ds(0, C)], wsem[u2]).wait()

        if tail or u4 >= 2:
            ww()
        else:
            pl.when(g >= 1)(ww)

        def row_body(i, a):
            a = list(a)
            for j in range(HL):
                sl = pl.ds(j * L, L)
                s = r1[u4][i, sl] + r2[u4][i, sl]
                mbuf[u2][i, sl] = s
                a[2 * j] = a[2 * j] + s
                a[2 * j + 1] = a[2 * j + 1] + s * s
            return tuple(a)

        accs = lax.fori_loop(0, C, row_body, accs)
        pltpu.async_copy(mbuf[u2], mraw_hbm.at[pl.ds(ebase + k * C, C)],
                         wsem[u2])
        pltpu.sync_copy(ones_v, cnt_sp.at[didx[u4]], add=True)
        if not tail:
            def pi():
                idx_load(k + 4, u4)

            if u4 == 0:
                pi()
            else:
                pl.when(g < G - 1)(pi)
        return accs

    for s in range(4):
        idx_load(s, s)
    idx_wait(0)
    gather(0)
    idx_wait(1)
    gather(1)

    def gbody(g, accs):
        for u in range(4):
            accs = step(4 * g + u, u, u % 2, g, accs)
        return accs

    accs = lax.fori_loop(
        0, G, gbody,
        tuple(jnp.zeros((L,), jnp.float32) for _ in range(2 * HL)))
    accs = step(NCHUNK - 1, 0, 0, G, accs, tail=True)
    pltpu.make_async_copy(mbuf[1], mraw_hbm.at[pl.ds(0, C)], wsem[1]).wait()
    pltpu.make_async_copy(mbuf[0], mraw_hbm.at[pl.ds(0, C)], wsem[0]).wait()

    for j in range(HL):
        stat_v[pl.ds(j * L, L)] = accs[2 * j]
    pltpu.sync_copy(stat_v, s1_hbm.at[wid])
    for j in range(HL):
        stat_v[pl.ds(j * L, L)] = accs[2 * j + 1]
    pltpu.sync_copy(stat_v, s2_hbm.at[wid])

    plsc.subcore_barrier()
    pltpu.sync_copy(cnt_sp.at[pl.ds(sid * RPT, RPT)],
                    cnt_hbm.at[cid, pl.ds(sid * RPT, RPT)])


@functools.partial(
    pl.kernel,
    out_type=[
        jax.ShapeDtypeStruct((E, H), jnp.float32),
        jax.ShapeDtypeStruct((NC, NP, H), jnp.float32),
    ],
    mesh=_mesh,
    scratch_types=[
        [pltpu.VMEM((C,), jnp.int32)] * 4,
        [pltpu.VMEM((C, H), jnp.float32)] * 2,
        [pltpu.VMEM((C, H), jnp.float32)] * 2,
        pltpu.VMEM((H,), jnp.float32),
        pltpu.VMEM((H,), jnp.float32),
        pltpu.VMEM_SHARED((NP, H), jnp.float32),
        [pltpu.SemaphoreType.DMA] * 4,
        [pltpu.SemaphoreType.DMA] * 2,
        [pltpu.SemaphoreType.DMA] * 2,
    ],
)
def _sc_pass2(dst_hbm, mraw_hbm, scale_hbm, shift_hbm,
              msg_hbm, agg_hbm,
              didx, min_v, mout_v, sc_v, sh_v, acc_sp,
              isem, rsem, wsem):
    cid = lax.axis_index("c")
    sid = lax.axis_index("s")
    wid = sid * NC + cid
    ebase = wid * EPW

    pltpu.sync_copy(scale_hbm, sc_v)
    pltpu.sync_copy(shift_hbm, sh_v)
    scs = [sc_v[pl.ds(j * L, L)] for j in range(HL)]
    shs = [sh_v[pl.ds(j * L, L)] for j in range(HL)]

    zero16 = jnp.zeros((L,), jnp.float32)

    def zrow(i, carry):
        for j in range(HL):
            mout_v[0][i, pl.ds(j * L, L)] = zero16
        return carry

    lax.fori_loop(0, C, zrow, 0)
    for t in range(RPT // C):
        pltpu.sync_copy(mout_v[0], acc_sp.at[pl.ds(sid * RPT + t * C, C)])
    plsc.subcore_barrier()

    def idx_load(k, s):
        pltpu.async_copy(dst_hbm.at[pl.ds(ebase + k * C, C)], didx[s],
                         isem[s])

    def idx_wait(s):
        pltpu.make_async_copy(dst_hbm.at[pl.ds(0, C)], didx[s],
                              isem[s]).wait()

    def mread(k, s):
        pltpu.async_copy(mraw_hbm.at[pl.ds(ebase + k * C, C)], min_v[s],
                         rsem[s])

    def mread_wait(s):
        pltpu.make_async_copy(mraw_hbm.at[pl.ds(0, C)], min_v[s],
                              rsem[s]).wait()

    def step(k, u4, u2, g, tail=False):
        mread_wait(u2)
        def ww():
            pltpu.make_async_copy(
                mout_v[u2], msg_hbm.at[pl.ds(0, C)], wsem[u2]).wait()

        if tail or u4 >= 2:
            ww()
        else:
            pl.when(g >= 1)(ww)

        def row_body(i, c2):
            for j in range(HL):
                sl = pl.ds(j * L, L)
                mout_v[u2][i, sl] = jnp.maximum(
                    min_v[u2][i, sl] * scs[j] + shs[j], 0.0)
            return c2

        lax.fori_loop(0, C, row_body, 0)
        idx_wait(u4)
        pltpu.async_copy(mout_v[u2], msg_hbm.at[pl.ds(ebase + k * C, C)],
                         wsem[u2])
        pltpu.sync_copy(mout_v[u2], acc_sp.at[didx[u4]], add=True)
        if not tail:
            s2 = (u4 + 2) % 4

            def pf():
                mread(k + 2, u2)
                idx_load(k + 2, s2)

            if u4 == 3:
                pl.when(g < G - 1)(pf)
            else:
                pf()

    for s in range(2):
        idx_load(s, s)
        mread(s, s)

    def gbody(g, carry):
        for u in range(4):
            step(4 * g + u, u, u % 2, g)
        return carry

    lax.fori_loop(0, G, gbody, 0)
    step(NCHUNK - 1, 0, 0, G, tail=True)
    pltpu.make_async_copy(mout_v[1], msg_hbm.at[pl.ds(0, C)], wsem[1]).wait()
    pltpu.make_async_copy(mout_v[0], msg_hbm.at[pl.ds(0, C)], wsem[0]).wait()

    plsc.subcore_barrier()
    pltpu.sync_copy(acc_sp.at[pl.ds(sid * RPT, RPT)],
                    agg_hbm.at[cid, pl.ds(sid * RPT, RPT)])


def _proj_body(x_ref, w1_ref, w2_ref, o1_ref, o2_ref):
    xb = x_ref[...]
    o1_ref[...] = jnp.dot(xb, w1_ref[...], preferred_element_type=jnp.float32)
    o2_ref[...] = jnp.dot(xb, w2_ref[...], preferred_element_type=jnp.float32)


def _project(x, w1t, w2t):
    nb = 10
    bs = N // nb
    return pl.pallas_call(
        _proj_body,
        grid=(nb,),
        in_specs=[
            pl.BlockSpec((bs, D), lambda i: (i, 0)),
            pl.BlockSpec((D, H), lambda i: (0, 0)),
            pl.BlockSpec((D, H), lambda i: (0, 0)),
        ],
        out_specs=[
            pl.BlockSpec((bs, H), lambda i: (i, 0)),
            pl.BlockSpec((bs, H), lambda i: (i, 0)),
        ],
        out_shape=[
            jax.ShapeDtypeStruct((N, H), jnp.float32),
            jax.ShapeDtypeStruct((N, H), jnp.float32),
        ],
    )(x, w1t, w2t)


def _mean_body(a0_ref, a1_ref, c0_ref, c1_ref, o_ref):
    cnt = jnp.maximum(c0_ref[0] + c1_ref[0], 1.0)
    o_ref[...] = (a0_ref[0] + a1_ref[0]) / cnt


def _finalize(agg, cnt3):
    nb = 10
    bs = N // nb
    return pl.pallas_call(
        _mean_body,
        grid=(nb,),
        in_specs=[
            pl.BlockSpec((1, bs, H), lambda i: (0, i, 0)),
            pl.BlockSpec((1, bs, H), lambda i: (1, i, 0)),
            pl.BlockSpec((1, bs, 1), lambda i: (0, i, 0)),
            pl.BlockSpec((1, bs, 1), lambda i: (1, i, 0)),
        ],
        out_specs=pl.BlockSpec((bs, H), lambda i: (i, 0)),
        out_shape=jax.ShapeDtypeStruct((N, H), jnp.float32),
    )(agg, agg, cnt3, cnt3)


@jax.jit
def kernel(x, edge_index, W, b, gamma, beta):
    src = edge_index[0].astype(jnp.int32)
    dst = edge_index[1].astype(jnp.int32)
    w1t = W[:, :D].T
    w2t = W[:, D:].T

    p1, p2 = _project(x, w1t, w2t)

    mraw, s1p, s2p, cntp = _sc_pass1(dst, src, p1, p2)

    s1 = jnp.sum(s1p, axis=0)
    s2 = jnp.sum(s2p, axis=0)
    mean_s = s1 / E
    var = s2 / E - mean_s * mean_s
    scale = gamma / jnp.sqrt(var + 1e-5)
    shift = beta - mean_s * scale

    msg, aggp = _sc_pass2(dst, mraw, scale, shift)

    out = _finalize(aggp, cntp[:, :, None])
    return out, msg


# speedup vs baseline: 8.8193x; 1.0583x over previous
import functools

import jax
import jax.numpy as jnp
from jax import lax
from jax.experimental import pallas as pl
from jax.experimental.pallas import tpu as pltpu
from jax.experimental.pallas import tpu_sc as plsc

N = 10000
E = 320000
D = 128
H = 128

NC = 2
NS = 16
NW = NC * NS
L = 16

NP = 10240
RPT = NP // NS
EPW = E // NW
C = 80
NCHUNK = EPW // C
G = NCHUNK // 4
HL = H // L

_mesh = plsc.VectorSubcoreMesh(
    core_axis_name="c", subcore_axis_name="s", num_cores=NC, num_subcores=NS)


@functools.partial(
    pl.kernel,
    out_type=[
        jax.ShapeDtypeStruct((E, H), jnp.float32),
        jax.ShapeDtypeStruct((NW, H), jnp.float32),
        jax.ShapeDtypeStruct((NW, H), jnp.float32),
        jax.ShapeDtypeStruct((NC, NP), jnp.float32),
    ],
    mesh=_mesh,
    scratch_types=[
        [pltpu.VMEM((C,), jnp.int32)] * 4,
        [pltpu.VMEM((C,), jnp.int32)] * 4,
        [pltpu.VMEM((C, H), jnp.float32)] * 4,
        [pltpu.VMEM((C, H), jnp.float32)] * 4,
        [pltpu.VMEM((C, H), jnp.float32)] * 2,
        pltpu.VMEM((C,), jnp.float32),
        pltpu.VMEM((H,), jnp.float32),
        pltpu.VMEM((RPT,), jnp.float32),
        pltpu.VMEM_SHARED((NP,), jnp.float32),
        [pltpu.SemaphoreType.DMA] * 4,
        [pltpu.SemaphoreType.DMA] * 4,
        [pltpu.SemaphoreType.DMA] * 2,
    ],
)
def _sc_pass1(dst_hbm, src_hbm, p1_hbm, p2_hbm,
              mraw_hbm, s1_hbm, s2_hbm, cnt_hbm,
              didx, sidx, r1, r2, mbuf, ones_v, stat_v, zbuf, cnt_sp,
              isem, gsem, wsem):
    cid = lax.axis_index("c")
    sid = lax.axis_index("s")
    wid = sid * NC + cid
    ebase = wid * EPW

    zero16 = jnp.zeros((L,), jnp.float32)
    for t in range(RPT // L):
        zbuf[pl.ds(t * L, L)] = zero16
    for t in range(C // L):
        ones_v[pl.ds(t * L, L)] = jnp.full((L,), 1.0, jnp.float32)
    pltpu.sync_copy(zbuf, cnt_sp.at[pl.ds(sid * RPT, RPT)])
    plsc.subcore_barrier()

    def idx_load(k, s):
        pltpu.async_copy(dst_hbm.at[pl.ds(ebase + k * C, C)], didx[s],
                         isem[s])
        pltpu.async_copy(src_hbm.at[pl.ds(ebase + k * C, C)], sidx[s],
                         isem[s])

    def idx_wait(s):
        pltpu.make_async_copy(dst_hbm.at[pl.ds(0, C)], didx[s],
                              isem[s]).wait()
        pltpu.make_async_copy(src_hbm.at[pl.ds(0, C)], sidx[s],
                              isem[s]).wait()

    def gather(s):
        pltpu.async_copy(p1_hbm.at[didx[s]], r1[s], gsem[s])
        pltpu.async_copy(p2_hbm.at[sidx[s]], r2[s], gsem[s])

    def gather_wait(s):
        pltpu.make_async_copy(p1_hbm.at[didx[s]], r1[s], gsem[s]).wait()
        pltpu.make_async_copy(p2_hbm.at[sidx[s]], r2[s], gsem[s]).wait()

    def step(k, u4, u2, g, accs, tail=False):
        gather_wait(u4)
        if not tail:
            s2 = (u4 + 2) % 4

            def pg():
                idx_wait(s2)
                gather(s2)

            if u4 == 3:
                pl.when(g < G - 1)(pg)
            else:
                pg()
        def ww():
            pltpu.make_async_copy(
                mbuf[u2], mraw_hbm.at[pl.ds(0, C)], wsem[u2]).wait()

        if tail or u4 >= 2:
            ww()
        else:
            pl.when(g >= 1)(ww)

        def row_body(i, a):
            a = list(a)
            for j in range(HL):
                sl = pl.ds(j * L, L)
                s = r1[u4][i, sl] + r2[u4][i, sl]
                mbuf[u2][i, sl] = s
                a[2 * j] = a[2 * j] + s
                a[2 * j + 1] = a[2 * j + 1] + s * s
            return tuple(a)

        accs = lax.fori_loop(0, C, row_body, accs)
        pltpu.async_copy(mbuf[u2], mraw_hbm.at[pl.ds(ebase + k * C, C)],
                         wsem[u2])
        pltpu.sync_copy(ones_v, cnt_sp.at[didx[u4]], add=True)
        if not tail:
            def pi():
                idx_load(k + 4, u4)

            if u4 == 0:
                pi()
            else:
                pl.when(g < G - 1)(pi)
        return accs

    for s in range(4):
        idx_load(s, s)
    idx_wait(0)
    gather(0)
    idx_wait(1)
    gather(1)

    def gbody(g, accs):
        for u in range(4):
            accs = step(4 * g + u, u, u % 2, g, accs)
        return accs

    accs = lax.fori_loop(
        0, G, gbody,
        tuple(jnp.zeros((L,), jnp.float32) for _ in range(2 * HL)))
    accs = step(NCHUNK - 1, 0, 0, G, accs, tail=True)
    pltpu.make_async_copy(mbuf[1], mraw_hbm.at[pl.ds(0, C)], wsem[1]).wait()
    pltpu.make_async_copy(mbuf[0], mraw_hbm.at[pl.ds(0, C)], wsem[0]).wait()

    for j in range(HL):
        stat_v[pl.ds(j * L, L)] = accs[2 * j]
    pltpu.sync_copy(stat_v, s1_hbm.at[wid])
    for j in range(HL):
        stat_v[pl.ds(j * L, L)] = accs[2 * j + 1]
    pltpu.sync_copy(stat_v, s2_hbm.at[wid])

    plsc.subcore_barrier()
    pltpu.sync_copy(cnt_sp.at[pl.ds(sid * RPT, RPT)],
                    cnt_hbm.at[cid, pl.ds(sid * RPT, RPT)])


@functools.partial(
    pl.kernel,
    out_type=[
        jax.ShapeDtypeStruct((E, H), jnp.float32),
        jax.ShapeDtypeStruct((NC, NP, H), jnp.float32),
    ],
    mesh=_mesh,
    scratch_types=[
        [pltpu.VMEM((C,), jnp.int32)] * 4,
        [pltpu.VMEM((C, H), jnp.float32)] * 2,
        [pltpu.VMEM((C, H), jnp.float32)] * 2,
        pltpu.VMEM((H,), jnp.float32),
        pltpu.VMEM((H,), jnp.float32),
        pltpu.VMEM_SHARED((NP, H), jnp.float32),
        [pltpu.SemaphoreType.DMA] * 4,
        [pltpu.SemaphoreType.DMA] * 2,
        [pltpu.SemaphoreType.DMA] * 2,
        [pltpu.SemaphoreType.DMA] * 2,
    ],
)
def _sc_pass2(dst_hbm, mraw_hbm, scale_hbm, shift_hbm,
              msg_hbm, agg_hbm,
              didx, min_v, mout_v, sc_v, sh_v, acc_sp,
              isem, rsem, wsem, ssem):
    cid = lax.axis_index("c")
    sid = lax.axis_index("s")
    wid = sid * NC + cid
    ebase = wid * EPW

    pltpu.sync_copy(scale_hbm, sc_v)
    pltpu.sync_copy(shift_hbm, sh_v)
    scs = [sc_v[pl.ds(j * L, L)] for j in range(HL)]
    shs = [sh_v[pl.ds(j * L, L)] for j in range(HL)]

    zero16 = jnp.zeros((L,), jnp.float32)

    def zrow(i, carry):
        for j in range(HL):
            mout_v[0][i, pl.ds(j * L, L)] = zero16
        return carry

    lax.fori_loop(0, C, zrow, 0)
    for t in range(RPT // C):
        pltpu.sync_copy(mout_v[0], acc_sp.at[pl.ds(sid * RPT + t * C, C)])
    plsc.subcore_barrier()

    def idx_load(k, s):
        pltpu.async_copy(dst_hbm.at[pl.ds(ebase + k * C, C)], didx[s],
                         isem[s])

    def idx_wait(s):
        pltpu.make_async_copy(dst_hbm.at[pl.ds(0, C)], didx[s],
                              isem[s]).wait()

    def mread(k, s):
        pltpu.async_copy(mraw_hbm.at[pl.ds(ebase + k * C, C)], min_v[s],
                         rsem[s])

    def mread_wait(s):
        pltpu.make_async_copy(mraw_hbm.at[pl.ds(0, C)], min_v[s],
                              rsem[s]).wait()

    def step(k, u4, u2, g, tail=False):
        mread_wait(u2)
        def ww():
            pltpu.make_async_copy(
                mout_v[u2], msg_hbm.at[pl.ds(0, C)], wsem[u2]).wait()
            pltpu.make_async_copy(
                mout_v[u2], acc_sp.at[didx[0]], ssem[u2]).wait()

        if tail or u4 >= 2:
            ww()
        else:
            pl.when(g >= 1)(ww)

        def row_body(i, c2):
            for j in range(HL):
                sl = pl.ds(j * L, L)
                mout_v[u2][i, sl] = jnp.maximum(
                    min_v[u2][i, sl] * scs[j] + shs[j], 0.0)
            return c2

        lax.fori_loop(0, C, row_body, 0)
        idx_wait(u4)
        pltpu.async_copy(mout_v[u2], msg_hbm.at[pl.ds(ebase + k * C, C)],
                         wsem[u2])
        pltpu.async_copy(mout_v[u2], acc_sp.at[didx[u4]], ssem[u2], add=True)
        if not tail:
            s2 = (u4 + 2) % 4

            def pf():
                mread(k + 2, u2)
                idx_load(k + 2, s2)

            if u4 == 3:
                pl.when(g < G - 1)(pf)
            else:
                pf()

    for s in range(2):
        idx_load(s, s)
        mread(s, s)

    def gbody(g, carry):
        for u in range(4):
            step(4 * g + u, u, u % 2, g)
        return carry

    lax.fori_loop(0, G, gbody, 0)
    step(NCHUNK - 1, 0, 0, G, tail=True)
    pltpu.make_async_copy(mout_v[1], msg_hbm.at[pl.ds(0, C)], wsem[1]).wait()
    pltpu.make_async_copy(mout_v[0], msg_hbm.at[pl.ds(0, C)], wsem[0]).wait()
    pltpu.make_async_copy(mout_v[1], acc_sp.at[didx[0]], ssem[1]).wait()
    pltpu.make_async_copy(mout_v[0], acc_sp.at[didx[0]], ssem[0]).wait()

    plsc.subcore_barrier()
    pltpu.sync_copy(acc_sp.at[pl.ds(sid * RPT, RPT)],
                    agg_hbm.at[cid, pl.ds(sid * RPT, RPT)])


def _proj_body(x_ref, w1_ref, w2_ref, o1_ref, o2_ref):
    xb = x_ref[...]
    o1_ref[...] = jnp.dot(xb, w1_ref[...], preferred_element_type=jnp.float32)
    o2_ref[...] = jnp.dot(xb, w2_ref[...], preferred_element_type=jnp.float32)


def _project(x, w1t, w2t):
    nb = 10
    bs = N // nb
    return pl.pallas_call(
        _proj_body,
        grid=(nb,),
        in_specs=[
            pl.BlockSpec((bs, D), lambda i: (i, 0)),
            pl.BlockSpec((D, H), lambda i: (0, 0)),
            pl.BlockSpec((D, H), lambda i: (0, 0)),
        ],
        out_specs=[
            pl.BlockSpec((bs, H), lambda i: (i, 0)),
            pl.BlockSpec((bs, H), lambda i: (i, 0)),
        ],
        out_shape=[
            jax.ShapeDtypeStruct((N, H), jnp.float32),
            jax.ShapeDtypeStruct((N, H), jnp.float32),
        ],
    )(x, w1t, w2t)


def _mean_body(a0_ref, a1_ref, c0_ref, c1_ref, o_ref):
    cnt = jnp.maximum(c0_ref[0] + c1_ref[0], 1.0)
    o_ref[...] = (a0_ref[0] + a1_ref[0]) / cnt


def _finalize(agg, cnt3):
    nb = 10
    bs = N // nb
    return pl.pallas_call(
        _mean_body,
        grid=(nb,),
        in_specs=[
            pl.BlockSpec((1, bs, H), lambda i: (0, i, 0)),
            pl.BlockSpec((1, bs, H), lambda i: (1, i, 0)),
            pl.BlockSpec((1, bs, 1), lambda i: (0, i, 0)),
            pl.BlockSpec((1, bs, 1), lambda i: (1, i, 0)),
        ],
        out_specs=pl.BlockSpec((bs, H), lambda i: (i, 0)),
        out_shape=jax.ShapeDtypeStruct((N, H), jnp.float32),
    )(agg, agg, cnt3, cnt3)


@jax.jit
def kernel(x, edge_index, W, b, gamma, beta):
    src = edge_index[0].astype(jnp.int32)
    dst = edge_index[1].astype(jnp.int32)
    w1t = W[:, :D].T
    w2t = W[:, D:].T

    p1, p2 = _project(x, w1t, w2t)

    mraw, s1p, s2p, cntp = _sc_pass1(dst, src, p1, p2)

    s1 = jnp.sum(s1p, axis=0)
    s2 = jnp.sum(s2p, axis=0)
    mean_s = s1 / E
    var = s2 / E - mean_s * mean_s
    scale = gamma / jnp.sqrt(var + 1e-5)
    shift = beta - mean_s * scale

    msg, aggp = _sc_pass2(dst, mraw, scale, shift)

    out = _finalize(aggp, cntp[:, :, None])
    return out, msg
